# baseline (speedup 1.0000x reference)
import functools

import jax
import jax.numpy as jnp
from jax import lax
from jax.experimental import pallas as pl
from jax.experimental.pallas import tpu as pltpu
from jax.experimental.pallas import tpu_sc as plsc

FIELDS = 26
DIM = 32
BATCH = 16384
N = BATCH * FIELDS
SUB = 128
NSUB = N // SUB
NPAT = 26 // 2
NC, NS, L = 2, 16, 16
NW = NC * NS
NSUB_W = NSUB // NW
GROUP = 8
ITERS = NSUB_W // GROUP

_mesh = plsc.VectorSubcoreMesh(core_axis_name="c", subcore_axis_name="s")


@functools.partial(
    pl.kernel,
    out_type=jax.ShapeDtypeStruct((NSUB, SUB, DIM), jnp.float32),
    mesh=_mesh,
    compiler_params=pltpu.CompilerParams(use_tc_tiling_on_sc=False),
    scratch_types=[
        pltpu.VMEM((NPAT, SUB), jnp.int32),
        pltpu.VMEM((GROUP, SUB), jnp.int32),
        pltpu.VMEM((GROUP, SUB, DIM), jnp.float32),
        pltpu.SemaphoreType.DMA,
    ],
)
def _buckle_gather(idx_hbm, pat_hbm, table_hbm, out_hbm,
                   pat_v, idx_v, rows_v, sem):
    wid = lax.axis_index("s") * NC + lax.axis_index("c")
    wbase = wid * NSUB_W
    pltpu.sync_copy(pat_hbm, pat_v)

    def iter_body(c, carry):
        cbase = wbase + c * GROUP
        pltpu.sync_copy(idx_hbm.at[pl.ds(cbase, GROUP)], idx_v)
        for j in range(GROUP):
            r = lax.rem(c * GROUP + j, NPAT)
            for s in range(SUB // L):
                sl = pl.ds(s * L, L)
                idx_v[j, sl] = idx_v[j, sl] + pat_v[r, sl]
        copies = [
            pltpu.async_copy(table_hbm.at[idx_v.at[j]], rows_v.at[j], sem)
            for j in range(GROUP)
        ]
        for cp in copies:
            cp.wait()
        pltpu.sync_copy(rows_v, out_hbm.at[pl.ds(cbase, GROUP)])
        return carry

    lax.fori_loop(0, ITERS, iter_body, 0)


def kernel(categorical_inputs, embedding_weight, offsets):
    idx = categorical_inputs.astype(jnp.int32).reshape(NSUB, SUB)
    fld = (24 * jnp.arange(NPAT, dtype=jnp.int32)[:, None]
           + jnp.arange(SUB, dtype=jnp.int32)[None, :]) % FIELDS
    pat = jnp.take(offsets[:FIELDS].astype(jnp.int32), fld, axis=0)
    out = _buckle_gather(idx, pat, embedding_weight)
    return out.reshape(BATCH, FIELDS, DIM)
